# baseline (speedup 1.0000x reference)
import jax
import jax.numpy as jnp
from jax import lax
from jax.experimental import pallas as pl
from jax.experimental.pallas import tpu as pltpu
from jax.experimental.pallas import tpu_sc as plsc

D = 64
ROWS_TOTAL = 4096 * 200
IDXW = 128
NROWS_IDX = ROWS_TOTAL // IDXW
NC, NS = 2, 16
NW = NC * NS
ROWS_PER_W = NROWS_IDX // NW
NSUB = 4
CHUNK = NSUB * IDXW
NCHUNK = ROWS_PER_W // NSUB
SCALE = 8.0


def _body(table_hbm, idx_hbm, out_hbm, idx_v, rows_v, sem):
    wid = lax.axis_index("s") * NC + lax.axis_index("c")
    row0 = wid * ROWS_PER_W

    def chunk_body(g, carry):
        base_idx_row = row0 + g * NSUB
        out_base = base_idx_row * IDXW
        pltpu.sync_copy(idx_hbm.at[pl.ds(base_idx_row, NSUB)], idx_v)
        descs = []
        for j in range(NSUB):
            descs.append(
                pltpu.async_copy(
                    table_hbm.at[idx_v.at[j]],
                    rows_v.at[pl.ds(j * IDXW, IDXW)],
                    sem,
                )
            )
        for d in descs:
            d.wait()

        def scale_row(r, c):
            for dcol in range(D // 16):
                sl = (r, pl.ds(dcol * 16, 16))
                rows_v[sl] = rows_v[sl] * SCALE
            return c

        lax.fori_loop(0, CHUNK, scale_row, 0, unroll=4)
        pltpu.sync_copy(rows_v, out_hbm.at[pl.ds(out_base, CHUNK)])
        return carry

    lax.fori_loop(0, NCHUNK, chunk_body, 0)


@jax.jit
def _emb(table, idx2d):
    mesh = plsc.VectorSubcoreMesh(core_axis_name="c", subcore_axis_name="s")
    return pl.kernel(
        _body,
        out_type=jax.ShapeDtypeStruct((ROWS_TOTAL, D), jnp.float32),
        mesh=mesh,
        compiler_params=pltpu.CompilerParams(use_tc_tiling_on_sc=False),
        scratch_types=[
            pltpu.VMEM((NSUB, IDXW), jnp.int32),
            pltpu.VMEM((CHUNK, D), jnp.float32),
            pltpu.SemaphoreType.DMA,
        ],
    )(table, idx2d)


def kernel(x, table):
    idx2d = x.reshape(NROWS_IDX, IDXW)
    out = _emb(table, idx2d)
    return out.reshape(x.shape[0], x.shape[1], D)


# speedup vs baseline: 1.0897x; 1.0897x over previous
import jax
import jax.numpy as jnp
from jax import lax
from jax.experimental import pallas as pl
from jax.experimental.pallas import tpu as pltpu
from jax.experimental.pallas import tpu_sc as plsc

D = 64
ROWS_TOTAL = 4096 * 200
IDXW = 128
NROWS_IDX = ROWS_TOTAL // IDXW
NC, NS = 2, 16
NW = NC * NS
ROWS_PER_W = NROWS_IDX // NW
NSUB = 2
CHUNK = NSUB * IDXW
NCHUNK = ROWS_PER_W // NSUB
NBUF = 4
SCALE = 8.0


def _body(table_hbm, idx_hbm, out_hbm, idx_bufs, row_bufs, gsems, ssems):
    wid = lax.axis_index("s") * NC + lax.axis_index("c")
    row0 = wid * ROWS_PER_W

    def fire_gathers(g, b):
        pltpu.sync_copy(idx_hbm.at[pl.ds(row0 + g * NSUB, NSUB)], idx_bufs[b])
        for j in range(NSUB):
            pltpu.async_copy(
                table_hbm.at[idx_bufs[b].at[j]],
                row_bufs[b].at[pl.ds(j * IDXW, IDXW)],
                gsems[b],
            )

    def wait_gathers(b):
        for j in range(NSUB):
            pltpu.make_async_copy(
                table_hbm.at[idx_bufs[b].at[j]],
                row_bufs[b].at[pl.ds(j * IDXW, IDXW)],
                gsems[b],
            ).wait()

    def wait_store(b):
        pltpu.make_async_copy(
            row_bufs[b], out_hbm.at[pl.ds(0, CHUNK)], ssems[b]
        ).wait()

    for b in range(NBUF - 1):
        fire_gathers(b, b)

    def chunk_iter(s, carry):
        for b in range(NBUF):
            g = s * NBUF + b
            wait_gathers(b)

            @plsc.parallel_loop(0, CHUNK, unroll=4)
            def scale_row(r):
                for dcol in range(D // 16):
                    sl = (r, pl.ds(dcol * 16, 16))
                    row_bufs[b][sl] = row_bufs[b][sl] * SCALE

            out_base = (row0 + g * NSUB) * IDXW
            pltpu.async_copy(
                row_bufs[b], out_hbm.at[pl.ds(out_base, CHUNK)], ssems[b]
            )

            bb = (b + NBUF - 1) % NBUF

            @pl.when(g + NBUF - 1 < NCHUNK)
            def _prime():
                @pl.when(g >= 1)
                def _drain_store():
                    wait_store(bb)

                fire_gathers(g + NBUF - 1, bb)

        return carry

    lax.fori_loop(0, NCHUNK // NBUF, chunk_iter, 0)

    for b in range(NBUF):
        wait_store(b)


@jax.jit
def _emb(table, idx2d):
    mesh = plsc.VectorSubcoreMesh(core_axis_name="c", subcore_axis_name="s")
    return pl.kernel(
        _body,
        out_type=jax.ShapeDtypeStruct((ROWS_TOTAL, D), jnp.float32),
        mesh=mesh,
        compiler_params=pltpu.CompilerParams(use_tc_tiling_on_sc=False),
        scratch_types=[
            [pltpu.VMEM((NSUB, IDXW), jnp.int32) for _ in range(NBUF)],
            [pltpu.VMEM((CHUNK, D), jnp.float32) for _ in range(NBUF)],
            [pltpu.SemaphoreType.DMA for _ in range(NBUF)],
            [pltpu.SemaphoreType.DMA for _ in range(NBUF)],
        ],
    )(table, idx2d)


def kernel(x, table):
    idx2d = x.reshape(NROWS_IDX, IDXW)
    out = _emb(table, idx2d)
    return out.reshape(x.shape[0], x.shape[1], D)
